# baseline (speedup 1.0000x reference)
import functools

import jax
import jax.numpy as jnp
from jax.experimental import pallas as pl
from jax.experimental.pallas import tpu as pltpu
from jax.experimental.pallas import tpu_sc as plsc

_B = 4096
_KD = 768
_PL = 8
_ED = 768
_NPOOL = 64
_NC = 2
_NS = 16
_NW = _NC * _NS
_BPW = _B // _NW

_BM = 512
_NB = _B // _BM


_WIN = 32
_NSP = 112
_NST = _BPW - _NSP
_CH = 8


def _gather_body(tab_ref, lab_ref, out_ref, shr_tab, lab_v, idx_v, buf0, buf1,
                 sem, stage_sem, gsem0, gsem1, ssem0, ssem1):
    c = jax.lax.axis_index("c")
    s = jax.lax.axis_index("s")
    wid = s * _NC + c
    base = pl.multiple_of(wid * _BPW, 8)

    rows_per_tile = _NPOOL // _NS
    soff = pl.multiple_of(s * rows_per_tile, 4)
    pltpu.async_copy(tab_ref.at[pl.ds(soff, rows_per_tile)],
                     shr_tab.at[pl.ds(soff, rows_per_tile)], stage_sem).wait()

    pltpu.sync_copy(lab_ref.at[pl.ds(base, _BPW)], lab_v)

    for i in range(_NST // 16):
        lv = lab_v[pl.ds(_NSP + i * 16, 16)]
        pv = jax.lax.rem(jax.lax.div(lv, jnp.int32(10)), jnp.int32(_NPOOL))
        idx_v[pl.ds(i * 16, 16)] = pv
    plsc.subcore_barrier()

    bufs = (buf0, buf1)
    gsems = (gsem0, gsem1)
    ssems = (ssem0, ssem1)
    n_ch = _NST // _CH
    gathers = [None] * n_ch
    scats = [None] * n_ch
    gathers[0] = pltpu.async_copy(
        tab_ref.at[idx_v.at[pl.ds(0, _CH)]], bufs[0], gsems[0])

    lane = jax.lax.broadcasted_iota(jnp.int32, (16,), 0)
    copies = []
    for i in range(_NSP // 16):
        lv = lab_v[pl.ds(i * 16, 16)]
        pv = jax.lax.rem(jax.lax.div(lv, jnp.int32(10)), jnp.int32(_NPOOL))
        for j in range(16):
            pij = jnp.sum(jnp.where(lane == j, pv, 0))
            if len(copies) >= _WIN:
                copies[len(copies) - _WIN].wait()
            copies.append(pltpu.async_copy(
                shr_tab.at[pl.ds(pij, 1)],
                out_ref.at[pl.ds(base + i * 16 + j, 1)], sem))
        g = i
        if g < n_ch:
            b = g % 2
            gathers[g].wait()
            if g + 1 < n_ch:
                ob = 1 - b
                if g >= 1:
                    scats[g - 1].wait()
                gathers[g + 1] = pltpu.async_copy(
                    tab_ref.at[idx_v.at[pl.ds((g + 1) * _CH, _CH)]],
                    bufs[ob], gsems[ob])
            scats[g] = pltpu.async_copy(
                bufs[b], out_ref.at[pl.ds(base + _NSP + g * _CH, _CH)],
                ssems[b])
    for g in range(_NSP // 16, n_ch):
        b = g % 2
        gathers[g].wait()
        if g + 1 < n_ch:
            ob = 1 - b
            if g >= 1:
                scats[g - 1].wait()
            gathers[g + 1] = pltpu.async_copy(
                tab_ref.at[idx_v.at[pl.ds((g + 1) * _CH, _CH)]],
                bufs[ob], gsems[ob])
        scats[g] = pltpu.async_copy(
            bufs[b], out_ref.at[pl.ds(base + _NSP + g * _CH, _CH)], ssems[b])
    for cp in copies[-_WIN:]:
        cp.wait()
    scats[n_ch - 2].wait()
    scats[n_ch - 1].wait()


def _sc_gather(tab, lab):
    mesh = plsc.VectorSubcoreMesh(core_axis_name="c", subcore_axis_name="s")
    run = functools.partial(
        pl.kernel, mesh=mesh,
        out_type=jax.ShapeDtypeStruct((_B, 1, _PL, _ED), jnp.float32),
        compiler_params=pltpu.CompilerParams(use_tc_tiling_on_sc=True, needs_layout_passes=False),
        scratch_types=[
            pltpu.VMEM_SHARED((_NPOOL, 1, _PL, _ED), jnp.float32),
            pltpu.VMEM((_BPW,), jnp.int32),
            pltpu.VMEM((_NST,), jnp.int32),
            pltpu.VMEM((_CH, 1, _PL, _ED), jnp.float32),
            pltpu.VMEM((_CH, 1, _PL, _ED), jnp.float32),
            pltpu.SemaphoreType.DMA,
            pltpu.SemaphoreType.DMA,
            pltpu.SemaphoreType.DMA,
            pltpu.SemaphoreType.DMA,
            pltpu.SemaphoreType.DMA,
            pltpu.SemaphoreType.DMA,
        ],
    )(_gather_body)
    return run(tab, lab)


def _loss_body(x_ref, ek_ref, lab_ref, out_ref, acc_ref):
    i = pl.program_id(0)

    @pl.when(i == 0)
    def _init():
        acc_ref[0] = 0.0
        acc_ref[1] = 0.0

    ek = ek_ref[...]
    nk = ek / jnp.maximum(jnp.sqrt(jnp.sum(ek * ek, axis=1, keepdims=True)),
                          1e-12)
    x = x_ref[...]
    q = x / jnp.maximum(jnp.sqrt(jnp.sum(x * x, axis=1, keepdims=True)),
                        1e-12)
    cos = jax.lax.dot_general(q, nk, (((1,), (1,)), ((), ())),
                              preferred_element_type=jnp.float32)
    s2 = jnp.sum(jnp.exp(cos))

    kk = jax.lax.dot_general(nk, nk, (((1,), (1,)), ((), ())),
                             preferred_element_type=jnp.float32)
    nrm = jnp.maximum(jnp.sqrt(jnp.sum(nk * nk, axis=1, keepdims=True)), 1e-8)
    outer = jax.lax.dot_general(nrm, nrm, (((1,), (1,)), ((), ())),
                                preferred_element_type=jnp.float32)
    r = jnp.sum(jnp.exp(1.0 - kk / outer), axis=1, keepdims=True)

    lab = lab_ref[0]
    pi = (lab // 10) % _NPOOL
    iota = jax.lax.broadcasted_iota(jnp.int32, (_NPOOL, 1), 0)
    s1 = jnp.sum(jnp.where(pi == iota, r, 0.0))

    acc_ref[0] = acc_ref[0] + s1
    acc_ref[1] = acc_ref[1] + s2

    @pl.when(i == _NB - 1)
    def _fin():
        s1t = acc_ref[0]
        s2t = acc_ref[1]
        val = s1t / (s2t + s1t) + 1e-6
        out_ref[...] = -jnp.log(jnp.full((1, 1), val, jnp.float32))


def _tc_loss(x, ek, lab, interpret=False):
    lab3 = lab.reshape(_NB, 1, _BM)
    return pl.pallas_call(
        _loss_body,
        grid=(_NB,),
        in_specs=[
            pl.BlockSpec((_BM, _KD), lambda i: (i, 0)),
            pl.BlockSpec((_NPOOL, _KD), lambda i: (0, 0)),
            pl.BlockSpec((1, 1, _BM), lambda i: (i, 0, 0)),
        ],
        out_specs=pl.BlockSpec((1, 1), lambda i: (0, 0)),
        out_shape=jax.ShapeDtypeStruct((1, 1), jnp.float32),
        scratch_shapes=[pltpu.SMEM((2,), jnp.float32)],
        interpret=interpret,
    )(x, ek, lab3)


def kernel(x_querry, l, x_block, label, train, e_k, e_p):
    lab = label.astype(jnp.int32)
    tab = e_p.reshape(_NPOOL, 1, _PL, _ED)
    P = _sc_gather(tab, lab)
    loss = _tc_loss(x_querry, e_k, lab)
    return P, loss.reshape(())


# speedup vs baseline: 1.0034x; 1.0034x over previous
import functools

import jax
import jax.numpy as jnp
from jax.experimental import pallas as pl
from jax.experimental.pallas import tpu as pltpu
from jax.experimental.pallas import tpu_sc as plsc

_B = 4096
_KD = 768
_PL = 8
_ED = 768
_NPOOL = 64
_NC = 2
_NS = 16
_NW = _NC * _NS
_BPW = _B // _NW

_BM = 512
_NB = _B // _BM


_WIN = 16
_NSP = 112
_NST = _BPW - _NSP
_CH = 8


def _gather_body(tab_ref, lab_ref, out_ref, shr_tab, lab_v, idx_v, buf0, buf1,
                 sem, stage_sem, gsem0, gsem1, ssem0, ssem1):
    c = jax.lax.axis_index("c")
    s = jax.lax.axis_index("s")
    wid = s * _NC + c
    base = pl.multiple_of(wid * _BPW, 8)

    rows_per_tile = _NPOOL // _NS
    soff = pl.multiple_of(s * rows_per_tile, 4)
    pltpu.async_copy(tab_ref.at[pl.ds(soff, rows_per_tile)],
                     shr_tab.at[pl.ds(soff, rows_per_tile)], stage_sem).wait()

    pltpu.sync_copy(lab_ref.at[pl.ds(base, _BPW)], lab_v)

    for i in range(_NST // 16):
        lv = lab_v[pl.ds(_NSP + i * 16, 16)]
        pv = jax.lax.rem(jax.lax.div(lv, jnp.int32(10)), jnp.int32(_NPOOL))
        idx_v[pl.ds(i * 16, 16)] = pv
    plsc.subcore_barrier()

    bufs = (buf0, buf1)
    gsems = (gsem0, gsem1)
    ssems = (ssem0, ssem1)
    n_ch = _NST // _CH
    gathers = [None] * n_ch
    scats = [None] * n_ch
    gathers[0] = pltpu.async_copy(
        tab_ref.at[idx_v.at[pl.ds(0, _CH)]], bufs[0], gsems[0])

    lane = jax.lax.broadcasted_iota(jnp.int32, (16,), 0)
    copies = []
    for i in range(_NSP // 16):
        lv = lab_v[pl.ds(i * 16, 16)]
        pv = jax.lax.rem(jax.lax.div(lv, jnp.int32(10)), jnp.int32(_NPOOL))
        for j in range(16):
            pij = jnp.sum(jnp.where(lane == j, pv, 0))
            if len(copies) >= _WIN:
                copies[len(copies) - _WIN].wait()
            copies.append(pltpu.async_copy(
                shr_tab.at[pl.ds(pij, 1)],
                out_ref.at[pl.ds(base + i * 16 + j, 1)], sem))
        g = i
        if g < n_ch:
            b = g % 2
            gathers[g].wait()
            if g + 1 < n_ch:
                ob = 1 - b
                if g >= 1:
                    scats[g - 1].wait()
                gathers[g + 1] = pltpu.async_copy(
                    tab_ref.at[idx_v.at[pl.ds((g + 1) * _CH, _CH)]],
                    bufs[ob], gsems[ob])
            scats[g] = pltpu.async_copy(
                bufs[b], out_ref.at[pl.ds(base + _NSP + g * _CH, _CH)],
                ssems[b])
    for g in range(_NSP // 16, n_ch):
        b = g % 2
        gathers[g].wait()
        if g + 1 < n_ch:
            ob = 1 - b
            if g >= 1:
                scats[g - 1].wait()
            gathers[g + 1] = pltpu.async_copy(
                tab_ref.at[idx_v.at[pl.ds((g + 1) * _CH, _CH)]],
                bufs[ob], gsems[ob])
        scats[g] = pltpu.async_copy(
            bufs[b], out_ref.at[pl.ds(base + _NSP + g * _CH, _CH)], ssems[b])
    for cp in copies[-_WIN:]:
        cp.wait()
    scats[n_ch - 2].wait()
    scats[n_ch - 1].wait()


def _sc_gather(tab, lab):
    mesh = plsc.VectorSubcoreMesh(core_axis_name="c", subcore_axis_name="s")
    run = functools.partial(
        pl.kernel, mesh=mesh,
        out_type=jax.ShapeDtypeStruct((_B, 1, _PL, _ED), jnp.float32),
        compiler_params=pltpu.CompilerParams(use_tc_tiling_on_sc=True, needs_layout_passes=False),
        scratch_types=[
            pltpu.VMEM_SHARED((_NPOOL, 1, _PL, _ED), jnp.float32),
            pltpu.VMEM((_BPW,), jnp.int32),
            pltpu.VMEM((_NST,), jnp.int32),
            pltpu.VMEM((_CH, 1, _PL, _ED), jnp.float32),
            pltpu.VMEM((_CH, 1, _PL, _ED), jnp.float32),
            pltpu.SemaphoreType.DMA,
            pltpu.SemaphoreType.DMA,
            pltpu.SemaphoreType.DMA,
            pltpu.SemaphoreType.DMA,
            pltpu.SemaphoreType.DMA,
            pltpu.SemaphoreType.DMA,
        ],
    )(_gather_body)
    return run(tab, lab)


def _loss_body(x_ref, ek_ref, lab_ref, out_ref, acc_ref):
    i = pl.program_id(0)

    @pl.when(i == 0)
    def _init():
        acc_ref[0] = 0.0
        acc_ref[1] = 0.0

    ek = ek_ref[...]
    nk = ek / jnp.maximum(jnp.sqrt(jnp.sum(ek * ek, axis=1, keepdims=True)),
                          1e-12)
    x = x_ref[...]
    q = x / jnp.maximum(jnp.sqrt(jnp.sum(x * x, axis=1, keepdims=True)),
                        1e-12)
    cos = jax.lax.dot_general(q, nk, (((1,), (1,)), ((), ())),
                              preferred_element_type=jnp.float32)
    s2 = jnp.sum(jnp.exp(cos))

    kk = jax.lax.dot_general(nk, nk, (((1,), (1,)), ((), ())),
                             preferred_element_type=jnp.float32)
    nrm = jnp.maximum(jnp.sqrt(jnp.sum(nk * nk, axis=1, keepdims=True)), 1e-8)
    outer = jax.lax.dot_general(nrm, nrm, (((1,), (1,)), ((), ())),
                                preferred_element_type=jnp.float32)
    r = jnp.sum(jnp.exp(1.0 - kk / outer), axis=1, keepdims=True)

    lab = lab_ref[0]
    pi = (lab // 10) % _NPOOL
    iota = jax.lax.broadcasted_iota(jnp.int32, (_NPOOL, 1), 0)
    s1 = jnp.sum(jnp.where(pi == iota, r, 0.0))

    acc_ref[0] = acc_ref[0] + s1
    acc_ref[1] = acc_ref[1] + s2

    @pl.when(i == _NB - 1)
    def _fin():
        s1t = acc_ref[0]
        s2t = acc_ref[1]
        val = s1t / (s2t + s1t) + 1e-6
        out_ref[...] = -jnp.log(jnp.full((1, 1), val, jnp.float32))


def _tc_loss(x, ek, lab, interpret=False):
    lab3 = lab.reshape(_NB, 1, _BM)
    return pl.pallas_call(
        _loss_body,
        grid=(_NB,),
        in_specs=[
            pl.BlockSpec((_BM, _KD), lambda i: (i, 0)),
            pl.BlockSpec((_NPOOL, _KD), lambda i: (0, 0)),
            pl.BlockSpec((1, 1, _BM), lambda i: (i, 0, 0)),
        ],
        out_specs=pl.BlockSpec((1, 1), lambda i: (0, 0)),
        out_shape=jax.ShapeDtypeStruct((1, 1), jnp.float32),
        scratch_shapes=[pltpu.SMEM((2,), jnp.float32)],
        interpret=interpret,
    )(x, ek, lab3)


def kernel(x_querry, l, x_block, label, train, e_k, e_p):
    lab = label.astype(jnp.int32)
    tab = e_p.reshape(_NPOOL, 1, _PL, _ED)
    P = _sc_gather(tab, lab)
    loss = _tc_loss(x_querry, e_k, lab)
    return P, loss.reshape(())


# speedup vs baseline: 1.0114x; 1.0080x over previous
import functools

import jax
import jax.numpy as jnp
from jax.experimental import pallas as pl
from jax.experimental.pallas import tpu as pltpu
from jax.experimental.pallas import tpu_sc as plsc

_B = 4096
_KD = 768
_PL = 8
_ED = 768
_NPOOL = 64
_NC = 2
_NS = 16
_NW = _NC * _NS
_BPW = _B // _NW

_BM = 512
_NB = _B // _BM


_WIN = 16
_NSP = 112
_NST = _BPW - _NSP
_CH = 8


def _gather_body(tab_ref, lab_ref, out_ref, shr_tab, lab_v, idx_v, buf0, buf1,
                 sem, stage_sem, gsem0, gsem1, ssem0, ssem1):
    c = jax.lax.axis_index("c")
    s = jax.lax.axis_index("s")
    wid = s * _NC + c
    base = pl.multiple_of(wid * _BPW, 8)

    rows_per_tile = _NPOOL // _NS
    soff = pl.multiple_of(s * rows_per_tile, 4)
    stage_cp = pltpu.async_copy(tab_ref.at[pl.ds(soff, rows_per_tile)],
                                shr_tab.at[pl.ds(soff, rows_per_tile)],
                                stage_sem)

    pltpu.sync_copy(lab_ref.at[pl.ds(base, _BPW)], lab_v)

    for i in range(_NST // 16):
        lv = lab_v[pl.ds(_NSP + i * 16, 16)]
        pv = jax.lax.rem(jax.lax.div(lv, jnp.int32(10)), jnp.int32(_NPOOL))
        idx_v[pl.ds(i * 16, 16)] = pv
    stage_cp.wait()
    plsc.subcore_barrier()

    bufs = (buf0, buf1)
    gsems = (gsem0, gsem1)
    ssems = (ssem0, ssem1)
    n_ch = _NST // _CH
    gathers = [None] * n_ch
    scats = [None] * n_ch
    gathers[0] = pltpu.async_copy(
        tab_ref.at[idx_v.at[pl.ds(0, _CH)]], bufs[0], gsems[0])

    lane = jax.lax.broadcasted_iota(jnp.int32, (16,), 0)
    copies = []
    for i in range(_NSP // 16):
        lv = lab_v[pl.ds(i * 16, 16)]
        pv = jax.lax.rem(jax.lax.div(lv, jnp.int32(10)), jnp.int32(_NPOOL))
        for j in range(16):
            pij = jnp.sum(jnp.where(lane == j, pv, 0))
            if len(copies) >= _WIN:
                copies[len(copies) - _WIN].wait()
            copies.append(pltpu.async_copy(
                shr_tab.at[pl.ds(pij, 1)],
                out_ref.at[pl.ds(base + i * 16 + j, 1)], sem))
        g = i
        if g < n_ch:
            b = g % 2
            gathers[g].wait()
            if g + 1 < n_ch:
                ob = 1 - b
                if g >= 1:
                    scats[g - 1].wait()
                gathers[g + 1] = pltpu.async_copy(
                    tab_ref.at[idx_v.at[pl.ds((g + 1) * _CH, _CH)]],
                    bufs[ob], gsems[ob])
            scats[g] = pltpu.async_copy(
                bufs[b], out_ref.at[pl.ds(base + _NSP + g * _CH, _CH)],
                ssems[b])
    for g in range(_NSP // 16, n_ch):
        b = g % 2
        gathers[g].wait()
        if g + 1 < n_ch:
            ob = 1 - b
            if g >= 1:
                scats[g - 1].wait()
            gathers[g + 1] = pltpu.async_copy(
                tab_ref.at[idx_v.at[pl.ds((g + 1) * _CH, _CH)]],
                bufs[ob], gsems[ob])
        scats[g] = pltpu.async_copy(
            bufs[b], out_ref.at[pl.ds(base + _NSP + g * _CH, _CH)], ssems[b])
    for cp in copies[-_WIN:]:
        cp.wait()
    scats[n_ch - 2].wait()
    scats[n_ch - 1].wait()


def _sc_gather(tab, lab):
    mesh = plsc.VectorSubcoreMesh(core_axis_name="c", subcore_axis_name="s")
    run = functools.partial(
        pl.kernel, mesh=mesh,
        out_type=jax.ShapeDtypeStruct((_B, 1, _PL, _ED), jnp.float32),
        compiler_params=pltpu.CompilerParams(use_tc_tiling_on_sc=True, needs_layout_passes=False),
        scratch_types=[
            pltpu.VMEM_SHARED((_NPOOL, 1, _PL, _ED), jnp.float32),
            pltpu.VMEM((_BPW,), jnp.int32),
            pltpu.VMEM((_NST,), jnp.int32),
            pltpu.VMEM((_CH, 1, _PL, _ED), jnp.float32),
            pltpu.VMEM((_CH, 1, _PL, _ED), jnp.float32),
            pltpu.SemaphoreType.DMA,
            pltpu.SemaphoreType.DMA,
            pltpu.SemaphoreType.DMA,
            pltpu.SemaphoreType.DMA,
            pltpu.SemaphoreType.DMA,
            pltpu.SemaphoreType.DMA,
        ],
    )(_gather_body)
    return run(tab, lab)


def _loss_body(x_ref, ek_ref, lab_ref, out_ref, acc_ref):
    i = pl.program_id(0)

    @pl.when(i == 0)
    def _init():
        acc_ref[0] = 0.0
        acc_ref[1] = 0.0

    ek = ek_ref[...]
    nk = ek / jnp.maximum(jnp.sqrt(jnp.sum(ek * ek, axis=1, keepdims=True)),
                          1e-12)
    x = x_ref[...]
    q = x / jnp.maximum(jnp.sqrt(jnp.sum(x * x, axis=1, keepdims=True)),
                        1e-12)
    cos = jax.lax.dot_general(q, nk, (((1,), (1,)), ((), ())),
                              preferred_element_type=jnp.float32)
    s2 = jnp.sum(jnp.exp(cos))

    kk = jax.lax.dot_general(nk, nk, (((1,), (1,)), ((), ())),
                             preferred_element_type=jnp.float32)
    nrm = jnp.maximum(jnp.sqrt(jnp.sum(nk * nk, axis=1, keepdims=True)), 1e-8)
    outer = jax.lax.dot_general(nrm, nrm, (((1,), (1,)), ((), ())),
                                preferred_element_type=jnp.float32)
    r = jnp.sum(jnp.exp(1.0 - kk / outer), axis=1, keepdims=True)

    lab = lab_ref[0]
    pi = (lab // 10) % _NPOOL
    iota = jax.lax.broadcasted_iota(jnp.int32, (_NPOOL, 1), 0)
    s1 = jnp.sum(jnp.where(pi == iota, r, 0.0))

    acc_ref[0] = acc_ref[0] + s1
    acc_ref[1] = acc_ref[1] + s2

    @pl.when(i == _NB - 1)
    def _fin():
        s1t = acc_ref[0]
        s2t = acc_ref[1]
        val = s1t / (s2t + s1t) + 1e-6
        out_ref[...] = -jnp.log(jnp.full((1, 1), val, jnp.float32))


def _tc_loss(x, ek, lab, interpret=False):
    lab3 = lab.reshape(_NB, 1, _BM)
    return pl.pallas_call(
        _loss_body,
        grid=(_NB,),
        in_specs=[
            pl.BlockSpec((_BM, _KD), lambda i: (i, 0)),
            pl.BlockSpec((_NPOOL, _KD), lambda i: (0, 0)),
            pl.BlockSpec((1, 1, _BM), lambda i: (i, 0, 0)),
        ],
        out_specs=pl.BlockSpec((1, 1), lambda i: (0, 0)),
        out_shape=jax.ShapeDtypeStruct((1, 1), jnp.float32),
        scratch_shapes=[pltpu.SMEM((2,), jnp.float32)],
        interpret=interpret,
    )(x, ek, lab3)


def kernel(x_querry, l, x_block, label, train, e_k, e_p):
    lab = label.astype(jnp.int32)
    tab = e_p.reshape(_NPOOL, 1, _PL, _ED)
    P = _sc_gather(tab, lab)
    loss = _tc_loss(x_querry, e_k, lab)
    return P, loss.reshape(())
